# baseline (speedup 1.0000x reference)
import functools

import jax
import jax.numpy as jnp
from jax import lax
from jax.experimental import pallas as pl
from jax.experimental.pallas import tpu as pltpu

_HI = lax.Precision.HIGHEST


def _prep_kernel(x_ref, w_ref, proj_ref, wh_ref, src_ref, dst_ref):
    x = x_ref[0]
    wh = jnp.dot(x, w_ref[...], preferred_element_type=jnp.float32)
    log2e = 1.4426950408889634
    sd = jnp.dot(wh, proj_ref[...], preferred_element_type=jnp.float32,
                 precision=_HI) * log2e
    wh_aug = jnp.concatenate(
        [wh, jnp.ones((wh.shape[0], 1), jnp.float32)], axis=1)
    wh_ref[0] = wh_aug.astype(jnp.bfloat16)
    src_ref[0] = sd[:, 0:1]
    dst_ref[0] = sd[:, 1:2]


def _main_kernel(adj_ref, src_ref, dst_ref, wh_ref,
                 wzr_ref, bzr_ref, wc_ref, bc_ref, whead_ref, bhead_ref,
                 mean_ref, logvar_ref, *, t_steps):
    adj = adj_ref[...]
    adj_bf = adj.astype(jnp.bfloat16)
    br_rows = adj.shape[0]
    h_dim = wc_ref.shape[1]
    h = jnp.zeros((br_rows, h_dim), jnp.float32)
    for t in range(t_steps):
        src_t = src_ref[t]
        dst_t = dst_ref[t]
        a = src_t + dst_t
        lr = jnp.maximum(a, 0.2 * a)
        p = jnp.exp2(lr).astype(jnp.bfloat16) * adj_bf
        wh_t = wh_ref[t]
        aggs = jnp.dot(p, wh_t, preferred_element_type=jnp.float32)
        s = aggs[:, h_dim:h_dim + 1]
        agg = aggs[:, :h_dim] / s
        x_t = jnp.where(agg > 0.0, agg, jnp.exp(jnp.minimum(agg, 0.0)) - 1.0)
        xh = jnp.concatenate([x_t, h], axis=1)
        zr = jax.nn.sigmoid(
            jnp.dot(xh, wzr_ref[...], preferred_element_type=jnp.float32)
            + bzr_ref[...])
        z = zr[:, :h_dim]
        r = zr[:, h_dim:]
        xrh = jnp.concatenate([x_t, r * h], axis=1)
        c = jnp.tanh(
            jnp.dot(xrh, wc_ref[...], preferred_element_type=jnp.float32)
            + bc_ref[...])
        h = (1.0 - z) * h + z * c
    head = jnp.dot(h, whead_ref[...], preferred_element_type=jnp.float32,
                   precision=_HI) + bhead_ref[...]
    mean_ref[...] = head[:, 0:1]
    logvar_ref[...] = head[:, 1:2]


def kernel(x_seq, adj, W_gat, a_src, a_dst, Wz, Uz, bz, Wr, Ur, br, Wc, Uc, bc,
           Wm, bm, Wl, bl):
    t_steps, n, f_in = x_seq.shape
    h_dim = W_gat.shape[1]
    br_rows = 512 if n % 512 == 0 else n

    wh, src, dst = pl.pallas_call(
        _prep_kernel,
        grid=(t_steps,),
        in_specs=[
            pl.BlockSpec((1, n, f_in), lambda t: (t, 0, 0)),
            pl.BlockSpec((f_in, h_dim), lambda t: (0, 0)),
            pl.BlockSpec((h_dim, 2), lambda t: (0, 0)),
        ],
        out_specs=[
            pl.BlockSpec((1, n, h_dim + 1), lambda t: (t, 0, 0)),
            pl.BlockSpec((1, n, 1), lambda t: (t, 0, 0)),
            pl.BlockSpec((1, n, 1), lambda t: (t, 0, 0)),
        ],
        out_shape=[
            jax.ShapeDtypeStruct((t_steps, n, h_dim + 1), jnp.bfloat16),
            jax.ShapeDtypeStruct((t_steps, n, 1), jnp.float32),
            jax.ShapeDtypeStruct((t_steps, n, 1), jnp.float32),
        ],
    )(x_seq, W_gat,
      jnp.concatenate([a_src.reshape(h_dim, 1), a_dst.reshape(h_dim, 1)],
                      axis=1))

    dst_rows = dst.reshape(t_steps, n)

    wzr = jnp.concatenate([jnp.concatenate([Wz, Wr], axis=1),
                           jnp.concatenate([Uz, Ur], axis=1)], axis=0)
    bzr = jnp.concatenate([bz, br]).reshape(1, 2 * h_dim)
    wc2 = jnp.concatenate([Wc, Uc], axis=0)
    whead = jnp.concatenate([Wm, Wl], axis=1)
    bhead = jnp.concatenate([bm, bl]).reshape(1, 2)

    num_blocks = n // br_rows
    full = lambda shape: pl.BlockSpec(shape, lambda i: (0,) * len(shape))
    mean, logvar = pl.pallas_call(
        functools.partial(_main_kernel, t_steps=t_steps),
        grid=(num_blocks,),
        in_specs=[
            pl.BlockSpec((br_rows, n), lambda i: (i, 0)),
            pl.BlockSpec((t_steps, br_rows, 1), lambda i: (0, i, 0)),
            full((t_steps, n)),
            full((t_steps, n, h_dim + 1)),
            full((2 * h_dim, 2 * h_dim)), full((1, 2 * h_dim)),
            full((2 * h_dim, h_dim)), full((1, h_dim)),
            full((h_dim, 2)), full((1, 2)),
        ],
        out_specs=[
            pl.BlockSpec((br_rows, 1), lambda i: (i, 0)),
            pl.BlockSpec((br_rows, 1), lambda i: (i, 0)),
        ],
        out_shape=[
            jax.ShapeDtypeStruct((n, 1), jnp.float32),
            jax.ShapeDtypeStruct((n, 1), jnp.float32),
        ],
        compiler_params=pltpu.CompilerParams(
            dimension_semantics=("arbitrary",),
            vmem_limit_bytes=100 * 1024 * 1024,
        ),
    )(adj, src, dst_rows, wh,
      wzr, bzr, wc2, bc.reshape(1, h_dim), whead, bhead)

    return (mean, logvar)


# speedup vs baseline: 1.0667x; 1.0667x over previous
import functools

import jax
import jax.numpy as jnp
from jax import lax
from jax.experimental import pallas as pl
from jax.experimental.pallas import tpu as pltpu

_HI = lax.Precision.HIGHEST


def _prep_kernel(x_ref, w_ref, proj_ref, wh_ref, src_ref, dst_ref):
    x = x_ref[0]
    wh = jnp.dot(x, w_ref[...], preferred_element_type=jnp.float32)
    log2e = 1.4426950408889634
    sd = jnp.dot(wh, proj_ref[...], preferred_element_type=jnp.float32,
                 precision=_HI) * log2e
    wh_aug = jnp.concatenate(
        [wh, jnp.ones((wh.shape[0], 1), jnp.float32)], axis=1)
    wh_ref[0] = wh_aug.astype(jnp.bfloat16)
    src_ref[0] = sd[:, 0:1]
    dst_ref[0] = sd[:, 1:2]


def _main_kernel(adj_ref, src_ref, dst_ref, wh_ref,
                 wzr_ref, bzr_ref, wc_ref, bc_ref, whead_ref, bhead_ref,
                 mean_ref, logvar_ref, *, t_steps):
    adj = adj_ref[...]
    adj_bf = adj.astype(jnp.bfloat16)
    br_rows = adj.shape[0]
    h_dim = wc_ref.shape[1]
    h = jnp.zeros((br_rows, h_dim), jnp.float32)
    for t in range(t_steps):
        src_t = src_ref[t]
        dst_t = dst_ref[t]
        a = src_t + dst_t
        lr = jnp.maximum(a, 0.2 * a)
        p = jnp.exp2(lr).astype(jnp.bfloat16) * adj_bf
        wh_t = wh_ref[t]
        aggs = jnp.dot(p, wh_t, preferred_element_type=jnp.float32)
        s = aggs[:, h_dim:h_dim + 1]
        agg = aggs[:, :h_dim] / s
        x_t = jnp.where(agg > 0.0, agg, jnp.exp(jnp.minimum(agg, 0.0)) - 1.0)
        xh = jnp.concatenate([x_t, h], axis=1)
        zr = jax.nn.sigmoid(
            jnp.dot(xh, wzr_ref[...], preferred_element_type=jnp.float32)
            + bzr_ref[...])
        z = zr[:, :h_dim]
        r = zr[:, h_dim:]
        xrh = jnp.concatenate([x_t, r * h], axis=1)
        c = jnp.tanh(
            jnp.dot(xrh, wc_ref[...], preferred_element_type=jnp.float32)
            + bc_ref[...])
        h = (1.0 - z) * h + z * c
    head = jnp.dot(h, whead_ref[...], preferred_element_type=jnp.float32,
                   precision=_HI) + bhead_ref[...]
    mean_ref[...] = head[:, 0:1]
    logvar_ref[...] = head[:, 1:2]


def kernel(x_seq, adj, W_gat, a_src, a_dst, Wz, Uz, bz, Wr, Ur, br, Wc, Uc, bc,
           Wm, bm, Wl, bl):
    t_steps, n, f_in = x_seq.shape
    h_dim = W_gat.shape[1]
    br_rows = 256 if n % 256 == 0 else n

    wh, src, dst = pl.pallas_call(
        _prep_kernel,
        grid=(t_steps,),
        in_specs=[
            pl.BlockSpec((1, n, f_in), lambda t: (t, 0, 0)),
            pl.BlockSpec((f_in, h_dim), lambda t: (0, 0)),
            pl.BlockSpec((h_dim, 2), lambda t: (0, 0)),
        ],
        out_specs=[
            pl.BlockSpec((1, n, h_dim + 1), lambda t: (t, 0, 0)),
            pl.BlockSpec((1, n, 1), lambda t: (t, 0, 0)),
            pl.BlockSpec((1, n, 1), lambda t: (t, 0, 0)),
        ],
        out_shape=[
            jax.ShapeDtypeStruct((t_steps, n, h_dim + 1), jnp.bfloat16),
            jax.ShapeDtypeStruct((t_steps, n, 1), jnp.float32),
            jax.ShapeDtypeStruct((t_steps, n, 1), jnp.float32),
        ],
    )(x_seq, W_gat,
      jnp.concatenate([a_src.reshape(h_dim, 1), a_dst.reshape(h_dim, 1)],
                      axis=1))

    dst_rows = dst.reshape(t_steps, n)

    wzr = jnp.concatenate([jnp.concatenate([Wz, Wr], axis=1),
                           jnp.concatenate([Uz, Ur], axis=1)], axis=0)
    bzr = jnp.concatenate([bz, br]).reshape(1, 2 * h_dim)
    wc2 = jnp.concatenate([Wc, Uc], axis=0)
    whead = jnp.concatenate([Wm, Wl], axis=1)
    bhead = jnp.concatenate([bm, bl]).reshape(1, 2)

    num_blocks = n // br_rows
    full = lambda shape: pl.BlockSpec(shape, lambda i: (0,) * len(shape))
    mean, logvar = pl.pallas_call(
        functools.partial(_main_kernel, t_steps=t_steps),
        grid=(num_blocks,),
        in_specs=[
            pl.BlockSpec((br_rows, n), lambda i: (i, 0)),
            pl.BlockSpec((t_steps, br_rows, 1), lambda i: (0, i, 0)),
            full((t_steps, n)),
            full((t_steps, n, h_dim + 1)),
            full((2 * h_dim, 2 * h_dim)), full((1, 2 * h_dim)),
            full((2 * h_dim, h_dim)), full((1, h_dim)),
            full((h_dim, 2)), full((1, 2)),
        ],
        out_specs=[
            pl.BlockSpec((br_rows, 1), lambda i: (i, 0)),
            pl.BlockSpec((br_rows, 1), lambda i: (i, 0)),
        ],
        out_shape=[
            jax.ShapeDtypeStruct((n, 1), jnp.float32),
            jax.ShapeDtypeStruct((n, 1), jnp.float32),
        ],
        compiler_params=pltpu.CompilerParams(
            dimension_semantics=("parallel",),
            vmem_limit_bytes=100 * 1024 * 1024,
        ),
    )(adj, src, dst_rows, wh,
      wzr, bzr, wc2, bc.reshape(1, h_dim), whead, bhead)

    return (mean, logvar)
